# Initial kernel scaffold; baseline (speedup 1.0000x reference)
import jax
import jax.numpy as jnp
from jax.experimental import pallas as pl


def kernel(atom_fea, nbr_fea, nbr_fea_idx, params):
    raise NotImplementedError("write your pallas kernel here")


# speedup vs baseline: 1.5718x; 1.5718x over previous
import functools

import jax
import jax.numpy as jnp
from jax import lax
from jax.experimental import pallas as pl
from jax.experimental.pallas import tpu as pltpu
from jax.experimental.pallas import tpu_sc as plsc

N = 10000
M = 32
AF = 64
NF = 41
E = N * M

R = 400
GRID = N // R
RB = R * M

NC = 2
NS = 16
NW = NC * NS
CH = 128
NCH = 79
EPAD = NW * NCH * CH


def _layernorm(x, scale, bias):
    mu = jnp.mean(x, axis=-1, keepdims=True)
    var = jnp.mean((x - mu) ** 2, axis=-1, keepdims=True)
    return (x - mu) * lax.rsqrt(var + 1e-6) * scale + bias


def _softplus(x):
    return jnp.maximum(x, 0.0) + jnp.log1p(jnp.exp(-jnp.abs(x)))


def _sigmoid(x):
    return 1.0 / (1.0 + jnp.exp(-x))


def _dot(a, b):
    return jnp.dot(a, b, preferred_element_type=jnp.float32)


def _sc_gather(table, idx3):
    mesh = plsc.VectorSubcoreMesh(core_axis_name="c", subcore_axis_name="s")

    @functools.partial(
        pl.kernel,
        mesh=mesh,
        out_type=jax.ShapeDtypeStruct((EPAD, AF), jnp.float32),
        scratch_types=[
            pltpu.VMEM((NCH, CH), jnp.int32),
            pltpu.VMEM((CH, AF), jnp.float32),
            pltpu.VMEM((CH, AF), jnp.float32),
            pltpu.SemaphoreType.DMA,
            pltpu.SemaphoreType.DMA,
        ],
        compiler_params=pltpu.CompilerParams(use_tc_tiling_on_sc=False),
    )
    def gather_kernel(table_hbm, idx_hbm, out_hbm, idx_v, buf0, buf1, sem0, sem1):
        wid = lax.axis_index("s") * NC + lax.axis_index("c")
        base = wid * (NCH * CH)
        pltpu.sync_copy(idx_hbm.at[wid], idx_v)
        cp0 = pltpu.async_copy(table_hbm.at[idx_v.at[0]], buf0, sem0)

        def body(p, _):
            c0 = 2 * p
            c1 = c0 + 1
            pltpu.make_async_copy(table_hbm.at[idx_v.at[c0]], buf0, sem0).wait()

            @pl.when(c1 < NCH)
            def _():
                pltpu.async_copy(table_hbm.at[idx_v.at[c1]], buf1, sem1)

            pltpu.sync_copy(buf0, out_hbm.at[pl.ds(base + c0 * CH, CH)])

            @pl.when(c1 < NCH)
            def _():
                pltpu.make_async_copy(table_hbm.at[idx_v.at[c1]], buf1, sem1).wait()

                @pl.when(c1 + 1 < NCH)
                def _():
                    pltpu.async_copy(table_hbm.at[idx_v.at[c1 + 1]], buf0, sem0)

                pltpu.sync_copy(buf1, out_hbm.at[pl.ds(base + c1 * CH, CH)])

            return 0

        del cp0
        lax.fori_loop(0, (NCH + 1) // 2, body, 0)

    return gather_kernel(table, idx3)


def _tc_embed(atom_fea, w_emb, b_emb, ln_s, ln_b, wn, bn, wc, bc):
    def body(af, we, be, ls, lb, wn_, bn_, wc_, bc_, x_o, y_o, pc_o):
        x = _dot(af[...], we[...]) + be[...]
        xn = _layernorm(x, ls[...], lb[...])
        x_o[...] = x
        y_o[...] = _dot(xn, wn_[...]) + bn_[...]
        pc_o[...] = _dot(xn, wc_[...]) + bc_[...]

    full = lambda shape: pl.BlockSpec(shape, lambda i: (0, 0))
    return pl.pallas_call(
        body,
        grid=(GRID,),
        in_specs=[
            pl.BlockSpec((R, atom_fea.shape[1]), lambda i: (i, 0)),
            full(w_emb.shape), full(b_emb.shape),
            full(ln_s.shape), full(ln_b.shape),
            full(wn.shape), full(bn.shape),
            full(wc.shape), full(bc.shape),
        ],
        out_specs=[
            pl.BlockSpec((R, AF), lambda i: (i, 0)),
            pl.BlockSpec((R, AF), lambda i: (i, 0)),
            pl.BlockSpec((R, AF), lambda i: (i, 0)),
        ],
        out_shape=[
            jax.ShapeDtypeStruct((N, AF), jnp.float32),
            jax.ShapeDtypeStruct((N, AF), jnp.float32),
            jax.ShapeDtypeStruct((N, AF), jnp.float32),
        ],
        compiler_params=pltpu.CompilerParams(
            dimension_semantics=("arbitrary",)),
    )(atom_fea, w_emb, b_emb, ln_s, ln_b, wn, bn, wc, bc)


def _edge_mix(pc, png, pe, wg, bg, wm, bm):
    pc3 = jnp.broadcast_to(pc[:, None, :], (R, M, AF)).reshape(RB, AF)
    inter = pc3 * png * pe
    g = _sigmoid(_dot(inter, wg) + bg)
    mm = _softplus(_dot(inter, wm) + bm)
    return (g * mm).reshape(R, M, AF).sum(axis=1)


def _tc_conv_step(x, pc, png, nf2, we, be, wg, bg, wm, bm,
                  ln_s, ln_b, wn, bn, wc, bc):
    def body(x_r, pc_r, png_r, nf_r, we_r, be_r, wg_r, bg_r, wm_r, bm_r,
             ls_r, lb_r, wn_r, bn_r, wc_r, bc_r, x_o, y_o, pc_o):
        pe = _dot(nf_r[...], we_r[...]) + be_r[...]
        s = _edge_mix(pc_r[...], png_r[...], pe, wg_r[...], bg_r[...],
                      wm_r[...], bm_r[...])
        xo = x_r[...] + s
        xn = _layernorm(xo, ls_r[...], lb_r[...])
        x_o[...] = xo
        y_o[...] = _dot(xn, wn_r[...]) + bn_r[...]
        pc_o[...] = _dot(xn, wc_r[...]) + bc_r[...]

    full = lambda shape: pl.BlockSpec(shape, lambda i: (0, 0))
    return pl.pallas_call(
        body,
        grid=(GRID,),
        in_specs=[
            pl.BlockSpec((R, AF), lambda i: (i, 0)),
            pl.BlockSpec((R, AF), lambda i: (i, 0)),
            pl.BlockSpec((RB, AF), lambda i: (i, 0)),
            pl.BlockSpec((RB, NF), lambda i: (i, 0)),
            full(we.shape), full(be.shape),
            full(wg.shape), full(bg.shape),
            full(wm.shape), full(bm.shape),
            full(ln_s.shape), full(ln_b.shape),
            full(wn.shape), full(bn.shape),
            full(wc.shape), full(bc.shape),
        ],
        out_specs=[
            pl.BlockSpec((R, AF), lambda i: (i, 0)),
            pl.BlockSpec((R, AF), lambda i: (i, 0)),
            pl.BlockSpec((R, AF), lambda i: (i, 0)),
        ],
        out_shape=[
            jax.ShapeDtypeStruct((N, AF), jnp.float32),
            jax.ShapeDtypeStruct((N, AF), jnp.float32),
            jax.ShapeDtypeStruct((N, AF), jnp.float32),
        ],
        compiler_params=pltpu.CompilerParams(
            dimension_semantics=("arbitrary",)),
    )(x, pc, png, nf2, we, be, wg, bg, wm, bm, ln_s, ln_b, wn, bn, wc, bc)


def _tc_conv_final(x, pc, png, nf2, we, be, wg, bg, wm, bm,
                   w1, b1, w2, b2, w3row, b3):
    def body(x_r, pc_r, png_r, nf_r, we_r, be_r, wg_r, bg_r, wm_r, bm_r,
             w1_r, b1_r, w2_r, b2_r, w3_r, b3_r, acc_o):
        pe = _dot(nf_r[...], we_r[...]) + be_r[...]
        s = _edge_mix(pc_r[...], png_r[...], pe, wg_r[...], bg_r[...],
                      wm_r[...], bm_r[...])
        xo = x_r[...] + s
        h = _softplus(_dot(xo, w1_r[...]) + b1_r[...])
        h2 = _softplus(_dot(h, w2_r[...]) + b2_r[...])
        part = (jnp.sum(h2 * w3_r[...]) + R * b3_r[0, 0]).reshape(1, 1)

        @pl.when(pl.program_id(0) == 0)
        def _():
            acc_o[...] = jnp.zeros((1, 1), jnp.float32)

        acc_o[...] += part

    full = lambda shape: pl.BlockSpec(shape, lambda i: (0, 0))
    out = pl.pallas_call(
        body,
        grid=(GRID,),
        in_specs=[
            pl.BlockSpec((R, AF), lambda i: (i, 0)),
            pl.BlockSpec((R, AF), lambda i: (i, 0)),
            pl.BlockSpec((RB, AF), lambda i: (i, 0)),
            pl.BlockSpec((RB, NF), lambda i: (i, 0)),
            full(we.shape), full(be.shape),
            full(wg.shape), full(bg.shape),
            full(wm.shape), full(bm.shape),
            full(w1.shape), full(b1.shape),
            full(w2.shape), full(b2.shape),
            full(w3row.shape), full(b3.shape),
        ],
        out_specs=pl.BlockSpec((1, 1), lambda i: (0, 0)),
        out_shape=jax.ShapeDtypeStruct((1, 1), jnp.float32),
        compiler_params=pltpu.CompilerParams(
            dimension_semantics=("arbitrary",)),
    )(x, pc, png, nf2, we, be, wg, bg, wm, bm, w1, b1, w2, b2, w3row, b3)
    return out[0, 0]


def kernel(atom_fea, nbr_fea, nbr_fea_idx, params):
    row = lambda v: v.reshape(1, -1).astype(jnp.float32)

    idx = nbr_fea_idx.reshape(E).astype(jnp.int32)
    idx3 = jnp.pad(idx, (0, EPAD - E)).reshape(NW, NCH, CH)
    nf2 = nbr_fea.reshape(E, NF)

    convs = params["convs"]
    c0 = convs[0]
    x, y, pc = _tc_embed(
        atom_fea, params["emb"][0], row(params["emb"][1]),
        row(c0["ln_scale"]), row(c0["ln_bias"]),
        c0["phi_n"][0], row(c0["phi_n"][1]),
        c0["phi_c"][0], row(c0["phi_c"][1]))

    for i, conv in enumerate(convs):
        png = _sc_gather(y, idx3)
        if i + 1 < len(convs):
            nxt = convs[i + 1]
            x, y, pc = _tc_conv_step(
                x, pc, png, nf2,
                conv["phi_e"][0], row(conv["phi_e"][1]),
                conv["gate"][0], row(conv["gate"][1]),
                conv["mag"][0], row(conv["mag"][1]),
                row(nxt["ln_scale"]), row(nxt["ln_bias"]),
                nxt["phi_n"][0], row(nxt["phi_n"][1]),
                nxt["phi_c"][0], row(nxt["phi_c"][1]))
        else:
            total = _tc_conv_final(
                x, pc, png, nf2,
                conv["phi_e"][0], row(conv["phi_e"][1]),
                conv["gate"][0], row(conv["gate"][1]),
                conv["mag"][0], row(conv["mag"][1]),
                params["ro1"][0], row(params["ro1"][1]),
                params["ro2"][0], row(params["ro2"][1]),
                params["ro3"][0].reshape(1, AF),
                params["ro3"][1].reshape(1, 1))
    return total


# speedup vs baseline: 4.4329x; 2.8203x over previous
import functools

import jax
import jax.numpy as jnp
from jax import lax
from jax.experimental import pallas as pl
from jax.experimental.pallas import tpu as pltpu
from jax.experimental.pallas import tpu_sc as plsc

N = 10000
M = 32
AF = 64
NF = 41
E = N * M

R = 400
GRID = N // R
RB = R * M
MP = M // 2
RP = R * MP

NC = 2
NS = 16
NW = NC * NS
CH = 128
KG = 4
NMB = 20
NCH = KG * NMB
MB = KG * CH
EPAD = NW * NCH * CH


def _layernorm(x, scale, bias):
    mu = jnp.mean(x, axis=-1, keepdims=True)
    var = jnp.mean((x - mu) ** 2, axis=-1, keepdims=True)
    return (x - mu) * lax.rsqrt(var + 1e-6) * scale + bias


def _softplus(x):
    return jnp.where(x > 20.0, x, jnp.log(1.0 + jnp.exp(x)))


def _sigmoid(x):
    return 0.5 * jnp.tanh(0.5 * x) + 0.5


def _dot(a, b):
    return jnp.dot(a, b, preferred_element_type=jnp.float32)


def _sc_gather(table, idx3):
    mesh = plsc.VectorSubcoreMesh(core_axis_name="c", subcore_axis_name="s")

    @functools.partial(
        pl.kernel,
        mesh=mesh,
        out_type=jax.ShapeDtypeStruct((EPAD, AF), jnp.float32),
        scratch_types=[
            pltpu.VMEM((NCH, CH), jnp.int32),
            pltpu.VMEM((MB, AF), jnp.float32),
            pltpu.VMEM((MB, AF), jnp.float32),
            pltpu.VMEM_SHARED((N, AF), jnp.float32),
            pltpu.SemaphoreType.DMA,
            pltpu.SemaphoreType.DMA,
            pltpu.SemaphoreType.DMA,
            pltpu.SemaphoreType.DMA,
        ],
        compiler_params=pltpu.CompilerParams(use_tc_tiling_on_sc=False),
    )
    def gather_kernel(table_hbm, idx_hbm, out_hbm, idx_v, buf_a, buf_b,
                      table_sp, gsem_a, gsem_b, osem_a, osem_b):
        wid = lax.axis_index("s") * NC + lax.axis_index("c")
        sid = lax.axis_index("s")
        base = wid * (NCH * CH)

        @pl.when(sid == 0)
        def _():
            pltpu.sync_copy(table_hbm, table_sp)

        pltpu.sync_copy(idx_hbm.at[wid], idx_v)
        plsc.subcore_barrier()

        def fire(m, buf, gsem):
            for k in range(KG):
                pltpu.async_copy(table_sp.at[idx_v.at[m * KG + k]],
                                 buf.at[pl.ds(k * CH, CH)], gsem)

        def gdrain(m, buf, gsem):
            for k in range(KG):
                pltpu.make_async_copy(table_sp.at[idx_v.at[m * KG + k]],
                                      buf.at[pl.ds(k * CH, CH)], gsem).wait()

        def out_start(m, buf, osem):
            pltpu.async_copy(buf, out_hbm.at[pl.ds(base + m * MB, MB)], osem)

        def out_wait(m, buf, osem):
            pltpu.make_async_copy(buf, out_hbm.at[pl.ds(base + m * MB, MB)],
                                  osem).wait()

        fire(0, buf_a, gsem_a)

        def body(p, _):
            m0 = 2 * p
            gdrain(m0, buf_a, gsem_a)

            @pl.when(p > 0)
            def _():
                out_wait(m0 - 1, buf_b, osem_b)

            fire(m0 + 1, buf_b, gsem_b)
            out_start(m0, buf_a, osem_a)

            gdrain(m0 + 1, buf_b, gsem_b)

            @pl.when(p < NMB // 2 - 1)
            def _():
                out_wait(m0, buf_a, osem_a)
                fire(m0 + 2, buf_a, gsem_a)

            out_start(m0 + 1, buf_b, osem_b)
            return 0

        lax.fori_loop(0, NMB // 2, body, 0)
        out_wait(NMB - 2, buf_a, osem_a)
        out_wait(NMB - 1, buf_b, osem_b)

    return gather_kernel(table, idx3)


def _tc_embed(atom_fea, w_emb, b_emb, ln_s, ln_b, wn, bn, wc, bc):
    def body(af, we, be, ls, lb, wn_, bn_, wc_, bc_, x_o, y_o, pc_o):
        x = _dot(af[...], we[...]) + be[...]
        xn = _layernorm(x, ls[...], lb[...])
        x_o[...] = x
        y_o[...] = _dot(xn, wn_[...]) + bn_[...]
        pc_o[...] = _dot(xn, wc_[...]) + bc_[...]

    full = lambda shape: pl.BlockSpec(shape, lambda i: (0, 0))
    return pl.pallas_call(
        body,
        grid=(GRID,),
        in_specs=[
            pl.BlockSpec((R, atom_fea.shape[1]), lambda i: (i, 0)),
            full(w_emb.shape), full(b_emb.shape),
            full(ln_s.shape), full(ln_b.shape),
            full(wn.shape), full(bn.shape),
            full(wc.shape), full(bc.shape),
        ],
        out_specs=[
            pl.BlockSpec((R, AF), lambda i: (i, 0)),
            pl.BlockSpec((R, AF), lambda i: (i, 0)),
            pl.BlockSpec((R, AF), lambda i: (i, 0)),
        ],
        out_shape=[
            jax.ShapeDtypeStruct((N, AF), jnp.float32),
            jax.ShapeDtypeStruct((N, AF), jnp.float32),
            jax.ShapeDtypeStruct((N, AF), jnp.float32),
        ],
        compiler_params=pltpu.CompilerParams(
            dimension_semantics=("arbitrary",)),
    )(atom_fea, w_emb, b_emb, ln_s, ln_b, wn, bn, wc, bc)


def _edge_mix(pc, png2, nf_r, we2, be2, wgm2, bgm2):
    pe2 = _dot(nf_r.reshape(RP, 2 * NF), we2) + be2
    pcc = jnp.concatenate([pc, pc], axis=1)
    pc2 = jnp.broadcast_to(pcc[:, None, :], (R, MP, 2 * AF)).reshape(RP, 2 * AF)
    inter2 = pc2 * png2 * pe2
    z2 = _dot(inter2, wgm2) + bgm2
    zg = jnp.concatenate([z2[:, 0:AF], z2[:, 2 * AF:3 * AF]], axis=1)
    zm = jnp.concatenate([z2[:, AF:2 * AF], z2[:, 3 * AF:4 * AF]], axis=1)
    gm = _sigmoid(zg) * _softplus(zm)
    s2 = gm.reshape(R, MP, 2 * AF).sum(axis=1)
    return s2[:, :AF] + s2[:, AF:]


def _tc_conv_step(x, pc, png1, nfp, we2, be2, wgm2, bgm2,
                  ln_s, ln_b, wn, bn, wc, bc):
    def body(x_r, pc_r, png_r, nf_r, we_r, be_r, wgm_r, bgm_r,
             ls_r, lb_r, wn_r, bn_r, wc_r, bc_r, x_o, y_o, pc_o):
        png2 = png_r[...].reshape(RP, 2 * AF)
        s = _edge_mix(pc_r[...], png2, nf_r[...], we_r[...], be_r[...],
                      wgm_r[...], bgm_r[...])
        xo = x_r[...] + s
        xn = _layernorm(xo, ls_r[...], lb_r[...])
        x_o[...] = xo
        y_o[...] = _dot(xn, wn_r[...]) + bn_r[...]
        pc_o[...] = _dot(xn, wc_r[...]) + bc_r[...]

    full = lambda shape: pl.BlockSpec(shape, lambda i: (0, 0))
    full1 = lambda shape: pl.BlockSpec(shape, lambda i: tuple(0 for _ in shape))
    return pl.pallas_call(
        body,
        grid=(GRID,),
        in_specs=[
            pl.BlockSpec((R, AF), lambda i: (i, 0)),
            pl.BlockSpec((R, AF), lambda i: (i, 0)),
            pl.BlockSpec((RB * AF,), lambda i: (i,)),
            pl.BlockSpec((R, MP, 2 * NF), lambda i: (i, 0, 0)),
            full(we2.shape), full(be2.shape),
            full(wgm2.shape), full(bgm2.shape),
            full(ln_s.shape), full(ln_b.shape),
            full(wn.shape), full(bn.shape),
            full(wc.shape), full(bc.shape),
        ],
        out_specs=[
            pl.BlockSpec((R, AF), lambda i: (i, 0)),
            pl.BlockSpec((R, AF), lambda i: (i, 0)),
            pl.BlockSpec((R, AF), lambda i: (i, 0)),
        ],
        out_shape=[
            jax.ShapeDtypeStruct((N, AF), jnp.float32),
            jax.ShapeDtypeStruct((N, AF), jnp.float32),
            jax.ShapeDtypeStruct((N, AF), jnp.float32),
        ],
        compiler_params=pltpu.CompilerParams(
            dimension_semantics=("arbitrary",)),
    )(x, pc, png1, nfp, we2, be2, wgm2, bgm2, ln_s, ln_b, wn, bn, wc, bc)


def _tc_conv_final(x, pc, png1, nfp, we2, be2, wgm2, bgm2,
                   w1, b1, w2, b2, w3row, b3):
    def body(x_r, pc_r, png_r, nf_r, we_r, be_r, wgm_r, bgm_r,
             w1_r, b1_r, w2_r, b2_r, w3_r, b3_r, acc_o):
        png2 = png_r[...].reshape(RP, 2 * AF)
        s = _edge_mix(pc_r[...], png2, nf_r[...], we_r[...], be_r[...],
                      wgm_r[...], bgm_r[...])
        xo = x_r[...] + s
        h = _softplus(_dot(xo, w1_r[...]) + b1_r[...])
        h2 = _softplus(_dot(h, w2_r[...]) + b2_r[...])
        part = (jnp.sum(h2 * w3_r[...]) + R * b3_r[0, 0]).reshape(1, 1)

        @pl.when(pl.program_id(0) == 0)
        def _():
            acc_o[...] = jnp.zeros((1, 1), jnp.float32)

        acc_o[...] += part

    full = lambda shape: pl.BlockSpec(shape, lambda i: (0, 0))
    out = pl.pallas_call(
        body,
        grid=(GRID,),
        in_specs=[
            pl.BlockSpec((R, AF), lambda i: (i, 0)),
            pl.BlockSpec((R, AF), lambda i: (i, 0)),
            pl.BlockSpec((RB * AF,), lambda i: (i,)),
            pl.BlockSpec((R, MP, 2 * NF), lambda i: (i, 0, 0)),
            full(we2.shape), full(be2.shape),
            full(wgm2.shape), full(bgm2.shape),
            full(w1.shape), full(b1.shape),
            full(w2.shape), full(b2.shape),
            full(w3row.shape), full(b3.shape),
        ],
        out_specs=pl.BlockSpec((1, 1), lambda i: (0, 0)),
        out_shape=jax.ShapeDtypeStruct((1, 1), jnp.float32),
        compiler_params=pltpu.CompilerParams(
            dimension_semantics=("arbitrary",)),
    )(x, pc, png1, nfp, we2, be2, wgm2, bgm2, w1, b1, w2, b2, w3row, b3)
    return out[0, 0]


def _blockdiag2(w):
    zw = jnp.zeros_like(w)
    return jnp.concatenate([jnp.concatenate([w, zw], axis=1),
                            jnp.concatenate([zw, w], axis=1)], axis=0)


def kernel(atom_fea, nbr_fea, nbr_fea_idx, params):
    row = lambda v: v.reshape(1, -1).astype(jnp.float32)
    row2 = lambda v: jnp.concatenate([row(v), row(v)], axis=1)

    idx = nbr_fea_idx.reshape(E).astype(jnp.int32)
    idx3 = jnp.pad(idx, (0, EPAD - E)).reshape(NW, NCH, CH)
    nfp = nbr_fea.reshape(N, MP, 2 * NF)

    convs = params["convs"]
    c0 = convs[0]
    x, y, pc = _tc_embed(
        atom_fea, params["emb"][0], row(params["emb"][1]),
        row(c0["ln_scale"]), row(c0["ln_bias"]),
        c0["phi_n"][0], row(c0["phi_n"][1]),
        c0["phi_c"][0], row(c0["phi_c"][1]))

    for i, conv in enumerate(convs):
        png1 = _sc_gather(y, idx3).reshape(EPAD * AF)
        we2 = _blockdiag2(conv["phi_e"][0])
        be2 = row2(conv["phi_e"][1])
        wgm2 = _blockdiag2(
            jnp.concatenate([conv["gate"][0], conv["mag"][0]], axis=1))
        bgm2 = row2(jnp.concatenate([conv["gate"][1], conv["mag"][1]]))
        if i + 1 < len(convs):
            nxt = convs[i + 1]
            x, y, pc = _tc_conv_step(
                x, pc, png1, nfp, we2, be2, wgm2, bgm2,
                row(nxt["ln_scale"]), row(nxt["ln_bias"]),
                nxt["phi_n"][0], row(nxt["phi_n"][1]),
                nxt["phi_c"][0], row(nxt["phi_c"][1]))
        else:
            total = _tc_conv_final(
                x, pc, png1, nfp, we2, be2, wgm2, bgm2,
                params["ro1"][0], row(params["ro1"][1]),
                params["ro2"][0], row(params["ro2"][1]),
                params["ro3"][0].reshape(1, AF),
                params["ro3"][1].reshape(1, 1))
    return total
